# Initial kernel scaffold; baseline (speedup 1.0000x reference)
import jax
import jax.numpy as jnp
from jax.experimental import pallas as pl


def kernel(X, pool_idx):
    raise NotImplementedError("write your pallas kernel here")


# speedup vs baseline: 5.3960x; 5.3960x over previous
import jax
import jax.numpy as jnp
from jax import lax
from jax.experimental import pallas as pl
from jax.experimental.pallas import tpu as pltpu
from jax.experimental.pallas import tpu_sc as plsc

N_NODES = 100000
D = 128
N_POOL = 200000
NC, NS = 2, 16
NW = NC * NS

XC = 200
NXCHUNK = N_NODES // XC
XK = (NXCHUNK + NW - 1) // NW
PC = 80
NCHUNK = N_POOL // PC
PK = (NCHUNK + NW - 1) // NW


def _sc_body(x_hbm, i0_hbm, i1_hbm, out_hbm,
             xbuf, idx0_v, idx1_v, a_v, b_v, sem0, sem1):
    w = lax.axis_index("s") * NC + lax.axis_index("c")

    def copy_body(k, carry):
        c = k * NW + w

        @pl.when(c < NXCHUNK)
        def _():
            base = c * XC
            pltpu.sync_copy(x_hbm.at[pl.ds(base, XC), :], xbuf)
            pltpu.sync_copy(xbuf, out_hbm.at[pl.ds(base, XC), :])

        return carry

    lax.fori_loop(0, XK, copy_body, 0)

    def pool_body(k, carry):
        c = k * NW + w

        @pl.when(c < NCHUNK)
        def _():
            base = c * PC
            pltpu.sync_copy(i0_hbm.at[pl.ds(base, PC)], idx0_v)
            pltpu.sync_copy(i1_hbm.at[pl.ds(base, PC)], idx1_v)
            cp0 = pltpu.async_copy(x_hbm.at[idx0_v], a_v, sem0)
            cp1 = pltpu.async_copy(x_hbm.at[idx1_v], b_v, sem1)
            cp0.wait()
            cp1.wait()

            def row(i, inner):
                for j in range(D // 16):
                    s = pl.ds(j * 16, 16)
                    a_v[i, s] = (a_v[i, s] + b_v[i, s]) * 0.5
                return inner

            lax.fori_loop(0, PC, row, 0)
            pltpu.sync_copy(a_v, out_hbm.at[pl.ds(N_NODES + base, PC), :])

        return carry

    lax.fori_loop(0, PK, pool_body, 0)


def kernel(X, pool_idx):
    idx0 = pool_idx[:, 0]
    idx1 = pool_idx[:, 1]
    mesh = plsc.VectorSubcoreMesh(core_axis_name="c", subcore_axis_name="s")
    f = pl.kernel(
        _sc_body,
        out_type=jax.ShapeDtypeStruct((N_NODES + N_POOL, D), jnp.float32),
        mesh=mesh,
        scratch_types=[
            pltpu.VMEM((XC, D), jnp.float32),
            pltpu.VMEM((PC,), jnp.int32),
            pltpu.VMEM((PC,), jnp.int32),
            pltpu.VMEM((PC, D), jnp.float32),
            pltpu.VMEM((PC, D), jnp.float32),
            pltpu.SemaphoreType.DMA,
            pltpu.SemaphoreType.DMA,
        ],
    )
    return f(X, idx0, idx1)
